# baseline (speedup 1.0000x reference)
import jax
import jax.numpy as jnp
from jax.experimental import pallas as pl
from jax.experimental.pallas import tpu as pltpu

_SCALE = 8.0


def _scaled_add_kernel(seqs_ref, pos_ref, out_ref):
    out_ref[...] = seqs_ref[...] * _SCALE + pos_ref[...]


def kernel(seqs, pos_emb):
    B, L, D = seqs.shape
    R, C = (L * D) // 128, 128
    x = seqs.reshape(B, R, C)
    p = pos_emb.reshape(1, R, C)
    blk = 128
    out = pl.pallas_call(
        _scaled_add_kernel,
        grid=(B // blk,),
        in_specs=[
            pl.BlockSpec((blk, R, C), lambda i: (i, 0, 0)),
            pl.BlockSpec((1, R, C), lambda i: (0, 0, 0)),
        ],
        out_specs=pl.BlockSpec((blk, R, C), lambda i: (i, 0, 0)),
        out_shape=jax.ShapeDtypeStruct((B, R, C), jnp.float32),
        compiler_params=pltpu.CompilerParams(
            dimension_semantics=("parallel",),
        ),
    )(x, p)
    return out.reshape(B, L, D)
